# baseline (speedup 1.0000x reference)
import functools

import jax
import jax.numpy as jnp
from jax import lax
from jax.experimental import pallas as pl
from jax.experimental.pallas import tpu as pltpu
from jax.experimental.pallas import tpu_sc as plsc

BATCH = 16384
MF_DIM = 32
MLP_HALF = 32


def _sc_gather(user_idx, item_idx, user_emb_mf, item_emb_mf, user_emb_mlp, item_emb_mlp):
    info = plsc.get_sparse_core_info()
    nc, ns = info.num_cores, info.num_subcores
    nw = nc * ns
    bpw = BATCH // nw

    mesh = plsc.VectorSubcoreMesh(core_axis_name="c", subcore_axis_name="s")
    row_t = jax.ShapeDtypeStruct((BATCH, MF_DIM), jnp.float32)

    @functools.partial(
        pl.kernel,
        mesh=mesh,
        compiler_params=pltpu.CompilerParams(use_tc_tiling_on_sc=False),
        out_type=[row_t, row_t, row_t, row_t],
        scratch_types=[
            pltpu.VMEM((bpw,), jnp.int32),
            pltpu.VMEM((bpw,), jnp.int32),
            pltpu.VMEM((bpw, MF_DIM), jnp.float32),
            pltpu.VMEM((bpw, MF_DIM), jnp.float32),
            pltpu.VMEM((bpw, MF_DIM), jnp.float32),
            pltpu.VMEM((bpw, MF_DIM), jnp.float32),
            pltpu.SemaphoreType.DMA,
            pltpu.SemaphoreType.DMA,
            pltpu.SemaphoreType.DMA,
            pltpu.SemaphoreType.DMA,
        ],
    )
    def gather_kernel(uidx_hbm, iidx_hbm, umf_hbm, imf_hbm, umlp_hbm, imlp_hbm,
                      o_umf, o_imf, o_umlp, o_imlp,
                      uidx_v, iidx_v, r_umf, r_imf, r_umlp, r_imlp,
                      s0, s1, s2, s3):
        wid = lax.axis_index("s") * nc + lax.axis_index("c")
        base = wid * bpw
        pltpu.sync_copy(uidx_hbm.at[pl.ds(base, bpw)], uidx_v)
        pltpu.sync_copy(iidx_hbm.at[pl.ds(base, bpw)], iidx_v)
        c0 = pltpu.async_copy(umf_hbm.at[uidx_v], r_umf, s0)
        c1 = pltpu.async_copy(imf_hbm.at[iidx_v], r_imf, s1)
        c2 = pltpu.async_copy(umlp_hbm.at[uidx_v], r_umlp, s2)
        c3 = pltpu.async_copy(imlp_hbm.at[iidx_v], r_imlp, s3)
        c0.wait()
        pltpu.sync_copy(r_umf, o_umf.at[pl.ds(base, bpw)])
        c1.wait()
        pltpu.sync_copy(r_imf, o_imf.at[pl.ds(base, bpw)])
        c2.wait()
        pltpu.sync_copy(r_umlp, o_umlp.at[pl.ds(base, bpw)])
        c3.wait()
        pltpu.sync_copy(r_imlp, o_imlp.at[pl.ds(base, bpw)])

    return gather_kernel(user_idx, item_idx, user_emb_mf, item_emb_mf,
                         user_emb_mlp, item_emb_mlp)


def _tc_body(umf, imf, umlp, imlp, w1u, w1i, b1, w2, b2, wp1, wp2, bp, out):
    gmf = umf[...] * imf[...]
    h1 = jnp.dot(umlp[...], w1u[...], preferred_element_type=jnp.float32)
    h1 = h1 + jnp.dot(imlp[...], w1i[...], preferred_element_type=jnp.float32)
    h1 = jnp.maximum(h1 + b1[...], 0.0)
    h2 = jnp.dot(h1, w2[...], preferred_element_type=jnp.float32)
    h2 = jnp.maximum(h2 + b2[...], 0.0)
    logits = jnp.sum(gmf * wp1[...], axis=1) + jnp.sum(h2 * wp2[...], axis=1)
    out[...] = jnp.clip(logits + bp[0, 0], -15.0, 15.0)


def _tc_mlp(umf, imf, umlp, imlp, W1, b1, W2, b2, Wp, bp):
    blk = 2048
    grid = BATCH // blk
    w1u = W1[:, :MLP_HALF].T
    w1i = W1[:, MLP_HALF:].T
    b1r = b1.reshape(1, -1)
    w2 = W2.T
    b2r = b2.reshape(1, -1)
    wp1 = Wp[:, :MF_DIM]
    wp2 = Wp[:, MF_DIM:]
    bpr = bp.reshape(1, 1)

    act_spec = pl.BlockSpec((blk, MF_DIM), lambda i: (i, 0))
    full = lambda shape: pl.BlockSpec(shape, lambda i: (0,) * len(shape))
    return pl.pallas_call(
        _tc_body,
        grid=(grid,),
        in_specs=[
            act_spec, act_spec, act_spec, act_spec,
            full((32, 32)), full((32, 32)), full((1, 32)),
            full((32, 16)), full((1, 16)),
            full((1, 32)), full((1, 16)), full((1, 1)),
        ],
        out_specs=pl.BlockSpec((blk,), lambda i: (i,)),
        out_shape=jax.ShapeDtypeStruct((BATCH,), jnp.float32),
    )(umf, imf, umlp, imlp, w1u, w1i, b1r, w2, b2r, wp1, wp2, bpr)


def kernel(user_idx, item_idx, user_emb_mf, item_emb_mf, user_emb_mlp, item_emb_mlp,
           W1, b1, W2, b2, Wp, bp):
    umf, imf, umlp, imlp = _sc_gather(
        user_idx.astype(jnp.int32), item_idx.astype(jnp.int32),
        user_emb_mf, item_emb_mf, user_emb_mlp, item_emb_mlp)
    return _tc_mlp(umf, imf, umlp, imlp, W1, b1, W2, b2, Wp, bp)
